# Initial kernel scaffold; baseline (speedup 1.0000x reference)
import jax
import jax.numpy as jnp
from jax.experimental import pallas as pl


def kernel(x_user, x_item, K_W_user, K_b_user, Q_W_user, Q_b_user, V_W_user, V_b_user, O_W_user, O_b_user, ln_g_user, ln_b_user, K_W_item, K_b_item, Q_W_item, Q_b_item, V_W_item, V_b_item, O_W_item, O_b_item, ln_g_item, ln_b_item, Wrel_clicks, Wrel_clicked_by, ei_clicks, ei_clicked_by):
    raise NotImplementedError("write your pallas kernel here")


# speedup vs baseline: 2.8663x; 2.8663x over previous
import functools
import math

import jax
import jax.numpy as jnp
from jax.experimental import pallas as pl
from jax.experimental.pallas import tpu as pltpu

IN = 128
OUT = 128
H = 4
HD = 32
SCALE = math.sqrt(HD)
EB = 1024
RB = 2000


def _proj_kernel(x_ref, w_ref, b_ref, o_ref):
    o_ref[...] = (
        jnp.dot(x_ref[...], w_ref[...], preferred_element_type=jnp.float32)
        + b_ref[...]
    )


def _proj(x, w, b):
    n = x.shape[0]
    return pl.pallas_call(
        _proj_kernel,
        grid=(n // RB,),
        in_specs=[
            pl.BlockSpec((RB, x.shape[1]), lambda i: (i, 0)),
            pl.BlockSpec(w.shape, lambda i: (0, 0)),
            pl.BlockSpec(b.shape, lambda i: (0, 0)),
        ],
        out_specs=pl.BlockSpec((RB, w.shape[1]), lambda i: (i, 0)),
        out_shape=jax.ShapeDtypeStruct((n, w.shape[1]), jnp.float32),
    )(x, w, b)


def _gather_kernel(idx_ref, tab_ref, o_ref):
    def body(i, carry):
        r = idx_ref[i]
        o_ref[pl.ds(i, 1), :] = tab_ref[pl.ds(r, 1), :]
        return carry

    jax.lax.fori_loop(0, EB, body, 0)


def _gather(idx, tab):
    e = idx.shape[0]
    w = tab.shape[1]
    return pl.pallas_call(
        _gather_kernel,
        grid=(e // EB,),
        in_specs=[
            pl.BlockSpec((EB,), lambda i: (i,), memory_space=pltpu.SMEM),
            pl.BlockSpec(tab.shape, lambda i: (0, 0)),
        ],
        out_specs=pl.BlockSpec((EB, w), lambda i: (i, 0)),
        out_shape=jax.ShapeDtypeStruct((e, w), jnp.float32),
    )(idx, tab)


def _edge_kernel(kv_ref, q_ref, w_ref, msg_ref, wn_ref, *, e_real):
    gid = pl.program_id(0) * EB + jax.lax.broadcasted_iota(
        jnp.int32, (EB, 1), 0)
    valid = (gid < e_real).astype(jnp.float32)
    wn = jnp.zeros((EB, 1), jnp.float32)
    for h in range(H):
        k = kv_ref[:, h * HD:(h + 1) * HD]
        v = kv_ref[:, OUT + h * HD:OUT + (h + 1) * HD]
        q = q_ref[:, h * HD:(h + 1) * HD]
        wh = w_ref[h * HD:(h + 1) * HD, :]
        qw = jnp.dot(q, wh, preferred_element_type=jnp.float32)
        score = jnp.sum(qw * k, axis=1, keepdims=True) * (1.0 / SCALE)
        a = jnp.exp(jnp.clip(score, -5.0, 5.0)) * valid
        msg_ref[:, h * HD:(h + 1) * HD] = v * a
        wn = wn + a
    wn_ref[...] = jnp.broadcast_to(wn * (1.0 / H), (EB, 8))


def _edge(kv, q, wrel, e_real):
    e = kv.shape[0]
    return pl.pallas_call(
        functools.partial(_edge_kernel, e_real=e_real),
        grid=(e // EB,),
        in_specs=[
            pl.BlockSpec((EB, 2 * OUT), lambda i: (i, 0)),
            pl.BlockSpec((EB, OUT), lambda i: (i, 0)),
            pl.BlockSpec(wrel.shape, lambda i: (0, 0)),
        ],
        out_specs=[
            pl.BlockSpec((EB, OUT), lambda i: (i, 0)),
            pl.BlockSpec((EB, 8), lambda i: (i, 0)),
        ],
        out_shape=[
            jax.ShapeDtypeStruct((e, OUT), jnp.float32),
            jax.ShapeDtypeStruct((e, 8), jnp.float32),
        ],
    )(kv, q, wrel)


def _scatter_kernel(idx_ref, msg_ref, wn_ref, acc_ref, nrm_ref):
    @pl.when(pl.program_id(0) == 0)
    def _init():
        acc_ref[...] = jnp.zeros_like(acc_ref)
        nrm_ref[...] = jnp.zeros_like(nrm_ref)

    def body(i, carry):
        d = idx_ref[i]
        acc_ref[pl.ds(d, 1), :] = acc_ref[pl.ds(d, 1), :] + msg_ref[pl.ds(i, 1), :]
        nrm_ref[pl.ds(d, 1), :] = nrm_ref[pl.ds(d, 1), :] + wn_ref[pl.ds(i, 1), :]
        return carry

    jax.lax.fori_loop(0, EB, body, 0)


def _scatter(idx, msg, wn, n):
    e = idx.shape[0]
    return pl.pallas_call(
        _scatter_kernel,
        grid=(e // EB,),
        in_specs=[
            pl.BlockSpec((EB,), lambda i: (i,), memory_space=pltpu.SMEM),
            pl.BlockSpec((EB, OUT), lambda i: (i, 0)),
            pl.BlockSpec((EB, 8), lambda i: (i, 0)),
        ],
        out_specs=[
            pl.BlockSpec((n, OUT), lambda i: (0, 0)),
            pl.BlockSpec((n, 8), lambda i: (0, 0)),
        ],
        out_shape=[
            jax.ShapeDtypeStruct((n, OUT), jnp.float32),
            jax.ShapeDtypeStruct((n, 8), jnp.float32),
        ],
    )(idx, msg, wn)


def _post_kernel(acc_ref, nrm_ref, x_ref, w_ref, b_ref, g_ref, bb_ref, o_ref):
    n = jnp.maximum(nrm_ref[:, 0:1], 1e-8)
    agg = acc_ref[...] / n
    h = (
        jnp.dot(agg, w_ref[...], preferred_element_type=jnp.float32)
        + b_ref[...]
        + x_ref[...]
    )
    mu = jnp.mean(h, axis=1, keepdims=True)
    var = jnp.mean((h - mu) ** 2, axis=1, keepdims=True)
    o_ref[...] = (h - mu) / jnp.sqrt(var + 1e-5) * g_ref[...] + bb_ref[...]


def _post(acc, nrm, x, w, b, g, bb):
    n = acc.shape[0]
    return pl.pallas_call(
        _post_kernel,
        grid=(n // RB,),
        in_specs=[
            pl.BlockSpec((RB, OUT), lambda i: (i, 0)),
            pl.BlockSpec((RB, 8), lambda i: (i, 0)),
            pl.BlockSpec((RB, IN), lambda i: (i, 0)),
            pl.BlockSpec(w.shape, lambda i: (0, 0)),
            pl.BlockSpec(b.shape, lambda i: (0, 0)),
            pl.BlockSpec(g.shape, lambda i: (0, 0)),
            pl.BlockSpec(bb.shape, lambda i: (0, 0)),
        ],
        out_specs=pl.BlockSpec((RB, OUT), lambda i: (i, 0)),
        out_shape=jax.ShapeDtypeStruct((n, OUT), jnp.float32),
    )(acc, nrm, x, w, b, g, bb)


def kernel(x_user, x_item, K_W_user, K_b_user, Q_W_user, Q_b_user, V_W_user,
           V_b_user, O_W_user, O_b_user, ln_g_user, ln_b_user, K_W_item,
           K_b_item, Q_W_item, Q_b_item, V_W_item, V_b_item, O_W_item,
           O_b_item, ln_g_item, ln_b_item, Wrel_clicks, Wrel_clicked_by,
           ei_clicks, ei_clicked_by):
    ei_c = ei_clicks.astype(jnp.int32)
    ei_cb = ei_clicked_by.astype(jnp.int32)

    wc_u = jnp.concatenate([K_W_user, V_W_user, Q_W_user], axis=1)
    bc_u = jnp.concatenate([K_b_user, V_b_user, Q_b_user])[None, :]
    wc_i = jnp.concatenate([K_W_item, V_W_item, Q_W_item], axis=1)
    bc_i = jnp.concatenate([K_b_item, V_b_item, Q_b_item])[None, :]
    proj_u = _proj(x_user, wc_u, bc_u)
    proj_i = _proj(x_item, wc_i, bc_i)
    kv_u, q_u = proj_u[:, :2 * OUT], proj_u[:, 2 * OUT:]
    kv_i, q_i = proj_i[:, :2 * OUT], proj_i[:, 2 * OUT:]

    wr_c = Wrel_clicks.reshape(H * HD, HD)
    wr_cb = Wrel_clicked_by.reshape(H * HD, HD)

    e_real = ei_c.shape[1]
    e_pad = ((e_real + EB - 1) // EB) * EB - e_real

    s, d = ei_c[0], ei_c[1]
    s = jnp.pad(s, (0, e_pad))
    d = jnp.pad(d, (0, e_pad))
    kv = _gather(s, kv_u)
    q = _gather(d, q_i)
    msg, wn = _edge(kv, q, wr_c, e_real)
    acc_i, nrm_i = _scatter(d, msg, wn, x_item.shape[0])

    s, d = ei_cb[0], ei_cb[1]
    s = jnp.pad(s, (0, e_pad))
    d = jnp.pad(d, (0, e_pad))
    kv = _gather(s, kv_i)
    q = _gather(d, q_u)
    msg, wn = _edge(kv, q, wr_cb, e_real)
    acc_u, nrm_u = _scatter(d, msg, wn, x_user.shape[0])

    out_u = _post(acc_u, nrm_u, x_user, O_W_user, O_b_user[None],
                  ln_g_user[None], ln_b_user[None])
    out_i = _post(acc_i, nrm_i, x_item, O_W_item, O_b_item[None],
                  ln_g_item[None], ln_b_item[None])
    return jnp.stack([out_u, out_i])
